# baseline (speedup 1.0000x reference)
import functools

import jax
import jax.numpy as jnp
from jax import lax
from jax.experimental import pallas as pl
from jax.experimental.pallas import tpu as pltpu, tpu_sc as plsc

_R = 5
_C = 5
_N = 20
_L = 16

_mesh = plsc.VectorSubcoreMesh(core_axis_name="c", subcore_axis_name="s")


@functools.partial(
    pl.kernel,
    out_type=jax.ShapeDtypeStruct((_R, _N), jnp.float32),
    mesh=_mesh,
    scratch_types=[
        pltpu.VMEM((_R, _L), jnp.float32),
        pltpu.VMEM((_C, _L), jnp.float32),
        pltpu.VMEM((_L,), jnp.float32),
        pltpu.VMEM((_L,), jnp.int32),
        pltpu.VMEM((_R, _N), jnp.float32),
    ],
)
def _sc_kernel(x_hbm, wt_hbm, b_hbm, idx_hbm, other_hbm, out_hbm,
               x_v, wt_v, b_v, idx_v, z_v):
    wid = lax.axis_index("s") * _mesh.num_cores + lax.axis_index("c")

    @pl.when(wid == 0)
    def _():
        pltpu.sync_copy(x_hbm, x_v)
        pltpu.sync_copy(wt_hbm, wt_v)
        pltpu.sync_copy(b_hbm, b_v)
        pltpu.sync_copy(idx_hbm, idx_v)
        pltpu.sync_copy(other_hbm, z_v)

        b_vec = b_v[pl.ds(0, _L)]
        ivec = idx_v[pl.ds(0, _L)]
        wt_rows = [wt_v[k, pl.ds(0, _L)] for k in range(_C)]
        col0 = lax.iota(jnp.int32, _L)
        col1 = col0 + (_N - _L)

        for i in range(_R):
            x_row = x_v[i, pl.ds(0, _L)]
            acc = b_vec
            for k in range(_C):
                acc = acc + jnp.full((_L,), x_row[k], jnp.float32) * wt_rows[k]
            z0 = z_v[i, pl.ds(0, _L)]
            z1 = z_v[i, pl.ds(_N - _L, _L)]
            for j in range(_C):
                cj = jnp.full((_L,), ivec[j], jnp.int32)
                yj = jnp.full((_L,), acc[j], jnp.float32)
                z0 = jnp.where(col0 == cj, yj, z0)
                z1 = jnp.where(col1 == cj, yj, z1)
            z_v[i, pl.ds(0, _L)] = z0
            z_v[i, pl.ds(_N - _L, _L)] = z1

        pltpu.sync_copy(z_v, out_hbm)


def kernel(x, indices, W, b, other):
    x_pad = jnp.zeros((_R, _L), jnp.float32).at[:, :_C].set(x)
    wt_pad = jnp.zeros((_C, _L), jnp.float32).at[:, :_C].set(W.T)
    b_pad = jnp.zeros((_L,), jnp.float32).at[:_C].set(b)
    idx_pad = jnp.full((_L,), -1, jnp.int32).at[:_C].set(
        indices.astype(jnp.int32))
    return _sc_kernel(x_pad, wt_pad, b_pad, idx_pad, other)


# speedup vs baseline: 1.0565x; 1.0565x over previous
import functools

import jax
import jax.numpy as jnp
from jax import lax
from jax.experimental import pallas as pl
from jax.experimental.pallas import tpu as pltpu, tpu_sc as plsc

_R = 5
_C = 5
_N = 20
_L = 16

_mesh = plsc.VectorSubcoreMesh(core_axis_name="c", subcore_axis_name="s",
                               num_cores=1)


@functools.partial(
    pl.kernel,
    out_type=jax.ShapeDtypeStruct((_R, _N), jnp.float32),
    mesh=_mesh,
    scratch_types=[
        pltpu.VMEM((_R, _L), jnp.float32),
        pltpu.VMEM((_C, _L), jnp.float32),
        pltpu.VMEM((_L,), jnp.float32),
        pltpu.VMEM((_L,), jnp.int32),
        pltpu.VMEM((_R, _N), jnp.float32),
    ],
)
def _sc_kernel(x_hbm, wt_hbm, b_hbm, idx_hbm, other_hbm, out_hbm,
               x_v, wt_v, b_v, idx_v, z_v):
    wid = lax.axis_index("s") * _mesh.num_cores + lax.axis_index("c")

    @pl.when(wid == 0)
    def _():
        pltpu.sync_copy(x_hbm, x_v)
        pltpu.sync_copy(wt_hbm, wt_v)
        pltpu.sync_copy(b_hbm, b_v)
        pltpu.sync_copy(idx_hbm, idx_v)
        pltpu.sync_copy(other_hbm, z_v)

        b_vec = b_v[pl.ds(0, _L)]
        ivec = idx_v[pl.ds(0, _L)]
        wt_rows = [wt_v[k, pl.ds(0, _L)] for k in range(_C)]
        col0 = lax.iota(jnp.int32, _L)
        col1 = col0 + (_N - _L)

        for i in range(_R):
            x_row = x_v[i, pl.ds(0, _L)]
            acc = b_vec
            for k in range(_C):
                acc = acc + jnp.full((_L,), x_row[k], jnp.float32) * wt_rows[k]
            z0 = z_v[i, pl.ds(0, _L)]
            z1 = z_v[i, pl.ds(_N - _L, _L)]
            for j in range(_C):
                cj = jnp.full((_L,), ivec[j], jnp.int32)
                yj = jnp.full((_L,), acc[j], jnp.float32)
                z0 = jnp.where(col0 == cj, yj, z0)
                z1 = jnp.where(col1 == cj, yj, z1)
            z_v[i, pl.ds(0, _L)] = z0
            z_v[i, pl.ds(_N - _L, _L)] = z1

        pltpu.sync_copy(z_v, out_hbm)


def kernel(x, indices, W, b, other):
    x_pad = jnp.zeros((_R, _L), jnp.float32).at[:, :_C].set(x)
    wt_pad = jnp.zeros((_C, _L), jnp.float32).at[:, :_C].set(W.T)
    b_pad = jnp.zeros((_L,), jnp.float32).at[:_C].set(b)
    idx_pad = jnp.full((_L,), -1, jnp.int32).at[:_C].set(
        indices.astype(jnp.int32))
    return _sc_kernel(x_pad, wt_pad, b_pad, idx_pad, other)


# speedup vs baseline: 1.2079x; 1.1433x over previous
import functools

import jax
import jax.numpy as jnp
from jax import lax
from jax.experimental import pallas as pl
from jax.experimental.pallas import tpu as pltpu, tpu_sc as plsc

_R = 5
_C = 5
_N = 20
_L = 16

_XO = 0
_WO = _XO + _R * _L
_BO = _WO + _C * _L
_IO = _BO + _L
_ZO = _IO + _L
_P = _ZO + 7 * _L

_mesh = plsc.VectorSubcoreMesh(core_axis_name="c", subcore_axis_name="s",
                               num_cores=1)


@functools.partial(
    pl.kernel,
    out_type=jax.ShapeDtypeStruct((_R, _N), jnp.float32),
    mesh=_mesh,
    scratch_types=[
        pltpu.VMEM((_P,), jnp.float32),
        pltpu.VMEM((_R, _N), jnp.float32),
    ],
)
def _sc_kernel(packed_hbm, out_hbm, p_v, z_v):
    wid = lax.axis_index("s") * _mesh.num_cores + lax.axis_index("c")

    @pl.when(wid == 0)
    def _():
        pltpu.sync_copy(packed_hbm, p_v)

        b_vec = p_v[pl.ds(_BO, _L)]
        ivec = p_v[pl.ds(_IO, _L)]
        wt_rows = [p_v[pl.ds(_WO + k * _L, _L)] for k in range(_C)]
        col0 = lax.iota(jnp.int32, _L).astype(jnp.float32)
        col1 = col0 + (_N - _L)

        for i in range(_R):
            x_row = p_v[pl.ds(_XO + i * _L, _L)]
            acc = b_vec
            for k in range(_C):
                acc = acc + jnp.full((_L,), x_row[k], jnp.float32) * wt_rows[k]
            z0 = p_v[pl.ds(_ZO + i * _N, _L)]
            z1 = p_v[pl.ds(_ZO + i * _N + _N - _L, _L)]
            for j in range(_C):
                cj = jnp.full((_L,), ivec[j], jnp.float32)
                yj = jnp.full((_L,), acc[j], jnp.float32)
                z0 = jnp.where(col0 == cj, yj, z0)
                z1 = jnp.where(col1 == cj, yj, z1)
            z_v[i, pl.ds(0, _L)] = z0
            z_v[i, pl.ds(_N - _L, _L)] = z1

        pltpu.sync_copy(z_v, out_hbm)


def kernel(x, indices, W, b, other):
    f32 = jnp.float32
    x_pad = jnp.zeros((_R, _L), f32).at[:, :_C].set(x)
    wt_pad = jnp.zeros((_C, _L), f32).at[:, :_C].set(W.T)
    b_pad = jnp.zeros((_L,), f32).at[:_C].set(b)
    idx_pad = jnp.full((_L,), -1.0, f32).at[:_C].set(
        indices.astype(f32))
    packed = jnp.concatenate([
        x_pad.reshape(-1),
        wt_pad.reshape(-1),
        b_pad,
        idx_pad,
        jnp.zeros((7 * _L,), f32).at[:_R * _N].set(other.reshape(-1)),
    ])
    return _sc_kernel(packed)
